# Initial kernel scaffold; baseline (speedup 1.0000x reference)
import jax
import jax.numpy as jnp
from jax.experimental import pallas as pl


def kernel(x):
    raise NotImplementedError("write your pallas kernel here")


# speedup vs baseline: 1.4632x; 1.4632x over previous
import functools

import jax
import jax.numpy as jnp
from jax import lax
from jax.experimental import pallas as pl
from jax.experimental.pallas import tpu as pltpu
from jax.experimental.pallas import tpu_sc as plsc

D = 512
TOTAL = 16384
SEG = 32
NSEG = TOTAL // SEG
LANES = 16

_mesh = plsc.VectorSubcoreMesh(core_axis_name="c", subcore_axis_name="s")
_NW = _mesh.num_cores * _mesh.num_subcores
_ROWS_PER_W = D // _NW


@functools.partial(
    pl.kernel,
    out_type=jax.ShapeDtypeStruct((D, NSEG), jnp.float32),
    mesh=_mesh,
    scratch_types=[
        pltpu.VMEM((TOTAL,), jnp.float32),
        pltpu.VMEM((NSEG,), jnp.float32),
    ],
    compiler_params=pltpu.CompilerParams(needs_layout_passes=False),
)
def _prod_at(x_hbm, out_hbm, row_buf, out_buf):
    wid = lax.axis_index("s") * _mesh.num_cores + lax.axis_index("c")
    lane = lax.iota(jnp.int32, LANES)
    base_idx = lane * SEG

    def row_body(r, carry):
        row = wid * _ROWS_PER_W + r
        pltpu.sync_copy(x_hbm.at[row], row_buf)

        def group_body(g, c2):
            idx0 = base_idx + g * (LANES * SEG)
            acc = plsc.load_gather(row_buf, [idx0])
            for k in range(1, SEG):
                acc = acc * plsc.load_gather(row_buf, [idx0 + k])
            plsc.store_scatter(out_buf, [lane + g * LANES], acc)
            return c2

        lax.fori_loop(0, NSEG // LANES, group_body, 0)
        pltpu.sync_copy(out_buf, out_hbm.at[row])
        return carry

    lax.fori_loop(0, _ROWS_PER_W, row_body, 0)


def kernel(x):
    return _prod_at(x)
